# baseline (speedup 1.0000x reference)
import functools

import jax
import jax.numpy as jnp
from jax import lax
from jax.experimental import pallas as pl
from jax.experimental.pallas import tpu as pltpu
from jax.experimental.pallas import tpu_sc as plsc

BATCH = 4096
HIST = 200
EMBED = 64
TOTAL = BATCH * HIST

NUM_CORES = 2
NUM_SUBCORES = 16
NUM_WORKERS = NUM_CORES * NUM_SUBCORES
ROWS_PER_WORKER = TOTAL // NUM_WORKERS

CHUNK = 1600
NUM_CHUNKS = ROWS_PER_WORKER // CHUNK

_mesh = plsc.VectorSubcoreMesh(core_axis_name="c", subcore_axis_name="s")


@functools.partial(
    pl.kernel,
    mesh=_mesh,
    compiler_params=pltpu.CompilerParams(use_tc_tiling_on_sc=False),
    out_type=jax.ShapeDtypeStruct((TOTAL, EMBED), jnp.float32),
    scratch_types=[
        pltpu.VMEM((CHUNK,), jnp.int32),
        pltpu.VMEM((CHUNK, EMBED), jnp.float32),
        pltpu.SemaphoreType.DMA,
    ],
)
def _gather_kernel(idx_hbm, table_hbm, out_hbm, idx_v, rows_v, sem):
    wid = lax.axis_index("s") * NUM_CORES + lax.axis_index("c")
    base = wid * ROWS_PER_WORKER

    def body(i, carry):
        off = base + i * CHUNK
        pltpu.sync_copy(idx_hbm.at[pl.ds(off, CHUNK)], idx_v)
        pltpu.async_copy(table_hbm.at[idx_v], rows_v, sem).wait()
        pltpu.sync_copy(rows_v, out_hbm.at[pl.ds(off, CHUNK)])
        return carry

    lax.fori_loop(0, NUM_CHUNKS, body, 0)


def kernel(input, table):
    idx = input.reshape(TOTAL).astype(jnp.int32)
    out = _gather_kernel(idx, table)
    return out.reshape(BATCH, HIST, EMBED)


# speedup vs baseline: 1.0046x; 1.0046x over previous
import functools

import jax
import jax.numpy as jnp
from jax import lax
from jax.experimental import pallas as pl
from jax.experimental.pallas import tpu as pltpu
from jax.experimental.pallas import tpu_sc as plsc

BATCH = 4096
HIST = 200
EMBED = 64
TOTAL = BATCH * HIST

NUM_CORES = 2
NUM_SUBCORES = 16
NUM_WORKERS = NUM_CORES * NUM_SUBCORES
ROWS_PER_WORKER = TOTAL // NUM_WORKERS

CHUNK = 400
NUM_CHUNKS = ROWS_PER_WORKER // CHUNK
NBUF = 4

_mesh = plsc.VectorSubcoreMesh(core_axis_name="c", subcore_axis_name="s")


@functools.partial(
    pl.kernel,
    mesh=_mesh,
    compiler_params=pltpu.CompilerParams(use_tc_tiling_on_sc=False),
    out_type=jax.ShapeDtypeStruct((TOTAL, EMBED), jnp.float32),
    scratch_types=[
        pltpu.VMEM((NUM_CHUNKS, CHUNK), jnp.int32),
        pltpu.VMEM((NBUF, CHUNK, EMBED), jnp.float32),
        pltpu.SemaphoreType.DMA((NBUF,)),
        pltpu.SemaphoreType.DMA((NBUF,)),
    ],
)
def _gather_kernel(idx_hbm, table_hbm, out_hbm, idx_v, rows_v, g_sem, w_sem):
    wid = lax.axis_index("s") * NUM_CORES + lax.axis_index("c")
    base = wid * ROWS_PER_WORKER

    def gather(i, slot):
        return pltpu.make_async_copy(
            table_hbm.at[idx_v.at[i]], rows_v.at[slot], g_sem.at[slot])

    def wb(i, slot):
        return pltpu.make_async_copy(
            rows_v.at[slot],
            out_hbm.at[pl.ds(base + i * CHUNK, CHUNK)],
            w_sem.at[slot])

    pltpu.sync_copy(idx_hbm.at[pl.ds(wid * NUM_CHUNKS, NUM_CHUNKS)], idx_v)

    gather(0, 0).start()
    gather(1, 1).start()
    gather(2, 2).start()
    gather(0, 0).wait()
    wb(0, 0).start()
    gather(3, 3).start()
    gather(1, 1).wait()
    wb(1, 1).start()
    wb(0, 0).wait()

    def body(g, carry):
        for b in range(NBUF):
            i = 2 + g * NBUF + b
            s = (2 + b) % NBUF
            gather(i + 2, (s + 2) % NBUF).start()
            gather(i, s).wait()
            wb(i, s).start()
            wb(i - 1, (s + NBUF - 1) % NBUF).wait()
        return carry

    lax.fori_loop(0, (NUM_CHUNKS - 4) // NBUF, body, 0)

    n = NUM_CHUNKS
    gather(n - 2, (n - 2) % NBUF).wait()
    wb(n - 2, (n - 2) % NBUF).start()
    wb(n - 3, (n - 3) % NBUF).wait()
    gather(n - 1, (n - 1) % NBUF).wait()
    wb(n - 1, (n - 1) % NBUF).start()
    wb(n - 2, (n - 2) % NBUF).wait()
    wb(n - 1, (n - 1) % NBUF).wait()


def kernel(input, table):
    idx = input.reshape(NUM_WORKERS * NUM_CHUNKS, CHUNK).astype(jnp.int32)
    out = _gather_kernel(idx, table)
    return out.reshape(BATCH, HIST, EMBED)


# speedup vs baseline: 1.2579x; 1.2521x over previous
import functools

import jax
import jax.numpy as jnp
from jax import lax
from jax.experimental import pallas as pl
from jax.experimental.pallas import tpu as pltpu
from jax.experimental.pallas import tpu_sc as plsc

BATCH = 4096
HIST = 200
EMBED = 64
TOTAL = BATCH * HIST

NUM_CORES = 2
NUM_SUBCORES = 16
NUM_WORKERS = NUM_CORES * NUM_SUBCORES
ROWS_PER_WORKER = TOTAL // NUM_WORKERS

CHUNK = 400
NUM_CHUNKS = ROWS_PER_WORKER // CHUNK
NBUF = 4

_mesh = plsc.VectorSubcoreMesh(core_axis_name="c", subcore_axis_name="s")


@functools.partial(
    pl.kernel,
    mesh=_mesh,
    compiler_params=pltpu.CompilerParams(use_tc_tiling_on_sc=False),
    out_type=jax.ShapeDtypeStruct((TOTAL, EMBED), jnp.float32),
    scratch_types=[
        pltpu.VMEM((NUM_CHUNKS, CHUNK), jnp.int32),
        pltpu.VMEM((NBUF, CHUNK, EMBED), jnp.float32),
        pltpu.SemaphoreType.DMA((NBUF,)),
        pltpu.SemaphoreType.DMA((NBUF,)),
    ],
)
def _gather_kernel(idx_hbm, table_hbm, out_hbm, idx_v, rows_v, g_sem, w_sem):
    wid = lax.axis_index("s") * NUM_CORES + lax.axis_index("c")
    base = wid * ROWS_PER_WORKER

    def gather(i, slot):
        return pltpu.make_async_copy(
            table_hbm.at[idx_v.at[i]], rows_v.at[slot], g_sem.at[slot])

    def wb(i, slot):
        return pltpu.make_async_copy(
            rows_v.at[slot],
            out_hbm.at[pl.ds(base + i * CHUNK, CHUNK)],
            w_sem.at[slot])

    pltpu.sync_copy(idx_hbm.at[pl.ds(wid * NUM_CHUNKS, NUM_CHUNKS)], idx_v)

    gather(0, 0).start()
    gather(1, 1).start()
    gather(2, 2).start()
    gather(0, 0).wait()
    wb(0, 0).start()
    gather(3, 3).start()
    gather(1, 1).wait()
    wb(1, 1).start()
    wb(0, 0).wait()

    def body(g, carry):
        for b in range(NBUF):
            i = 2 + g * NBUF + b
            s = (2 + b) % NBUF
            gather(i + 2, (s + 2) % NBUF).start()
            gather(i, s).wait()
            wb(i, s).start()
            wb(i - 1, (s + NBUF - 1) % NBUF).wait()
        return carry

    lax.fori_loop(0, (NUM_CHUNKS - 4) // NBUF, body, 0)

    n = NUM_CHUNKS
    gather(n - 2, (n - 2) % NBUF).wait()
    wb(n - 2, (n - 2) % NBUF).start()
    wb(n - 3, (n - 3) % NBUF).wait()
    gather(n - 1, (n - 1) % NBUF).wait()
    wb(n - 1, (n - 1) % NBUF).start()
    wb(n - 2, (n - 2) % NBUF).wait()
    wb(n - 1, (n - 1) % NBUF).wait()


def kernel(input, table):
    from jax.experimental.layout import Format, Layout, with_layout_constraint
    idx = input.reshape(NUM_WORKERS * NUM_CHUNKS, CHUNK).astype(jnp.int32)
    tbl = with_layout_constraint(
        table, Layout(major_to_minor=(0, 1), tiling=((8,),)))
    out = _gather_kernel(idx, tbl)
    return out.reshape(BATCH, HIST, EMBED)
